# baseline (speedup 1.0000x reference)
import functools

import jax
import jax.numpy as jnp
from jax import lax
from jax.experimental import pallas as pl
from jax.experimental.pallas import tpu as pltpu
from jax.experimental.pallas import tpu_sc as plsc

_TARGET_RATE = 0.5
_INT_MIN = -2147483648


def _build_mask_sc(imp_flat, B, N, k):
    info = plsc.get_sparse_core_info()
    NC, NS = info.num_cores, info.num_subcores
    WPB = (NC * NS) // B
    CH = N // WPB
    NV = CH // 16
    U = 8 if NV % 8 == 0 else 1
    NG = NV // U
    ROW = 16
    RPW = 3
    mesh = plsc.VectorSubcoreMesh(core_axis_name="c", subcore_axis_name="s")

    @functools.partial(
        pl.kernel,
        mesh=mesh,
        compiler_params=pltpu.CompilerParams(needs_layout_passes=False),
        out_type=jax.ShapeDtypeStruct((B * N,), jnp.float32),
        scratch_types=[
            pltpu.VMEM((CH,), jnp.float32),
            pltpu.VMEM((CH,), jnp.int32),
            pltpu.VMEM((CH,), jnp.float32),
            pltpu.VMEM((RPW * ROW,), jnp.int32),
            pltpu.VMEM((WPB * RPW * ROW,), jnp.int32),
            pltpu.VMEM_SHARED((2 * NS * RPW * ROW,), jnp.int32),
        ],
    )
    def sc_mask(imp_hbm, out_hbm, x_v, key_v, out_v, stage_v, team_v, counts_sm):
        c = lax.axis_index("c")
        s = lax.axis_index("s")
        batch = c * (B // NC) + s // WPB
        slot = s % WPB
        team_lo = (s // WPB) * WPB
        base = batch * N + slot * CH

        pltpu.sync_copy(imp_hbm.at[pl.ds(base, CH)], x_v)

        def keys_body(g, carry):
            for u in range(U):
                i = g * U + u
                bits = lax.bitcast_convert_type(x_v[pl.ds(i * 16, 16)],
                                                jnp.int32)
                key_v[pl.ds(i * 16, 16)] = jnp.where(
                    bits >= 0, bits, jnp.int32(_INT_MIN) - bits)
            return carry

        lax.fori_loop(0, NG, keys_body, jnp.int32(0))

        one = jnp.int32(1)
        zero16 = jnp.zeros((16,), jnp.int32)

        def publish(parity, vecs):
            for j, vec in enumerate(vecs):
                stage_v[pl.ds(j * ROW, ROW)] = vec
            off = (parity * NS + s) * (RPW * ROW)
            pltpu.sync_copy(stage_v, counts_sm.at[pl.ds(off, RPW * ROW)])
            plsc.subcore_barrier()
            toff = (parity * NS + team_lo) * (RPW * ROW)
            pltpu.sync_copy(counts_sm.at[pl.ds(toff, WPB * RPW * ROW)], team_v)

        def team_sum(j):
            def body(r, acc):
                return acc + team_v[pl.ds(r * (RPW * ROW) + j * ROW, ROW)]
            return jnp.sum(lax.fori_loop(0, WPB, body, zero16))

        def radix_body(t, basev):
            shift = jnp.int32(30) - 2 * t
            cand1 = basev + (one << shift)
            cand2 = basev + (jnp.int32(2) << shift)
            cand3 = basev + (jnp.int32(3) << shift)

            def scan(g, accs):
                a1, a2, a3 = accs
                for u in range(U):
                    kv = key_v[pl.ds((g * U + u) * 16, 16)]
                    a1 = a1 + jnp.where(kv >= cand1, one, 0)
                    a2 = a2 + jnp.where(kv >= cand2, one, 0)
                    a3 = a3 + jnp.where(kv >= cand3, one, 0)
                return a1, a2, a3

            a1, a2, a3 = lax.fori_loop(0, NG, scan, (zero16, zero16, zero16))
            publish(t % 2, [a1, a2, a3])
            t1, t2, t3 = team_sum(0), team_sum(1), team_sum(2)
            return jnp.where(
                t3 >= k, cand3,
                jnp.where(t2 >= k, cand2, jnp.where(t1 >= k, cand1, basev)))

        v = lax.fori_loop(0, 16, radix_body, jnp.int32(_INT_MIN))

        def count_scan(g, accs):
            ag, at_ = accs
            for u in range(U):
                kv = key_v[pl.ds((g * U + u) * 16, 16)]
                ag = ag + jnp.where(kv > v, one, 0)
                at_ = at_ + jnp.where(kv == v, one, 0)
            return ag, at_

        accg, acct = lax.fori_loop(0, NG, count_scan, (zero16, zero16))
        publish(0, [accg, acct])
        r_need = jnp.int32(k) - team_sum(0)
        tie_local = jnp.sum(acct)

        def prefix_body(rr, acc):
            rowsum = jnp.sum(team_v[pl.ds(rr * (RPW * ROW) + ROW, ROW)])
            return acc + jnp.where(rr < slot, rowsum, jnp.int32(0))

        tie_before = lax.fori_loop(0, WPB, prefix_body, jnp.int32(0))
        q = jnp.minimum(jnp.maximum(r_need - tie_before, jnp.int32(0)),
                        tie_local)

        fone, fzero = jnp.float32(1.0), jnp.float32(0.0)

        def write_plain(_):
            def body(g, carry):
                for u in range(U):
                    i = g * U + u
                    kv = key_v[pl.ds(i * 16, 16)]
                    out_v[pl.ds(i * 16, 16)] = jnp.where(kv > v, fone, fzero)
                return carry
            return lax.fori_loop(0, NG, body, jnp.int32(0))

        def write_all_ties(_):
            def body(g, carry):
                for u in range(U):
                    i = g * U + u
                    kv = key_v[pl.ds(i * 16, 16)]
                    out_v[pl.ds(i * 16, 16)] = jnp.where(kv >= v, fone, fzero)
                return carry
            return lax.fori_loop(0, NG, body, jnp.int32(0))

        def write_cumsum(_):
            def body(i, run):
                kv = key_v[pl.ds(i * 16, 16)]
                tie = kv == v
                csum = lax.cumsum(jnp.where(tie, one, 0))
                accept = tie & ((run + csum) <= q)
                out_v[pl.ds(i * 16, 16)] = jnp.where(
                    (kv > v) | accept, fone, fzero)
                return run + jnp.max(csum)
            return lax.fori_loop(0, NV, body, jnp.int32(0))

        _ = lax.cond(
            q == 0, write_plain,
            lambda _: lax.cond(q == tie_local, write_all_ties,
                               write_cumsum, 0),
            0)

        pltpu.sync_copy(out_v, out_hbm.at[pl.ds(base, CH)])

    return sc_mask(imp_flat)


def _sc_multiply(spikes_flat, mask, T, B, C, N):
    info = plsc.get_sparse_core_info()
    NC, NS = info.num_cores, info.num_subcores
    WPB = (NC * NS) // B
    RPW = (T * C) // WPB
    HALF = N // 2
    U = 8
    NGRP = HALF // 16 // U
    mesh = plsc.VectorSubcoreMesh(core_axis_name="c", subcore_axis_name="s")

    @functools.partial(
        pl.kernel,
        mesh=mesh,
        compiler_params=pltpu.CompilerParams(needs_layout_passes=False),
        out_type=jax.ShapeDtypeStruct((T * B * C * N,), jnp.float32),
        scratch_types=[
            pltpu.VMEM((N,), jnp.float32),
            pltpu.VMEM((HALF,), jnp.float32),
        ],
    )
    def sc_mul(spikes_hbm, mask_hbm, out_hbm, mask_v, dbuf):
        c = lax.axis_index("c")
        s = lax.axis_index("s")
        batch = c * (B // NC) + s // WPB
        wslot = s % WPB

        pltpu.sync_copy(mask_hbm.at[pl.ds(batch * N, N)], mask_v)

        def row_body(j, carry):
            jj = wslot * RPW + j
            t = jj // C
            cch = jj % C
            rbase = (((t * B) + batch) * C + cch) * N
            for h in range(2):
                off = rbase + h * HALF
                pltpu.sync_copy(spikes_hbm.at[pl.ds(off, HALF)], dbuf)

                def mul_body(g, cc):
                    for u in range(U):
                        i = g * U + u
                        dbuf[pl.ds(i * 16, 16)] = (
                            dbuf[pl.ds(i * 16, 16)]
                            * mask_v[pl.ds(h * HALF + i * 16, 16)])
                    return cc

                lax.fori_loop(0, NGRP, mul_body, jnp.int32(0))
                pltpu.sync_copy(dbuf, out_hbm.at[pl.ds(off, HALF)])
            return carry

        lax.fori_loop(0, RPW, row_body, jnp.int32(0))

    return sc_mul(spikes_flat, mask)


def kernel(spikes, importance, training):
    T, B, C, H, W = spikes.shape
    N = H * W
    k = max(1, int(_TARGET_RATE * N))
    mask = _build_mask_sc(importance.reshape(B * N), B, N, k)
    out = _sc_multiply(spikes.reshape(T * B * C * N), mask, T, B, C, N)
    return out.reshape(T, B, C, H, W)


# speedup vs baseline: 1.1695x; 1.1695x over previous
import functools

import jax
import jax.numpy as jnp
from jax import lax
from jax.experimental import pallas as pl
from jax.experimental.pallas import tpu as pltpu
from jax.experimental.pallas import tpu_sc as plsc

_TARGET_RATE = 0.5
_INT_MIN = -2147483648


def _build_mask_sc(imp_flat, B, N, k):
    info = plsc.get_sparse_core_info()
    NC, NS = info.num_cores, info.num_subcores
    WPB = (NC * NS) // B
    CH = N // WPB
    NV = CH // 16
    U = 8 if NV % 8 == 0 else 1
    NG = NV // U
    ROW = 16
    RPW = 3
    mesh = plsc.VectorSubcoreMesh(core_axis_name="c", subcore_axis_name="s")

    @functools.partial(
        pl.kernel,
        mesh=mesh,
        compiler_params=pltpu.CompilerParams(needs_layout_passes=False),
        out_type=jax.ShapeDtypeStruct((B * N,), jnp.float32),
        scratch_types=[
            pltpu.VMEM((CH,), jnp.float32),
            pltpu.VMEM((CH,), jnp.int32),
            pltpu.VMEM((CH,), jnp.float32),
            pltpu.VMEM((RPW * ROW,), jnp.int32),
            pltpu.VMEM((WPB * RPW * ROW,), jnp.int32),
            pltpu.VMEM_SHARED((2 * NS * RPW * ROW,), jnp.int32),
        ],
    )
    def sc_mask(imp_hbm, out_hbm, x_v, key_v, out_v, stage_v, team_v, counts_sm):
        c = lax.axis_index("c")
        s = lax.axis_index("s")
        batch = c * (B // NC) + s // WPB
        slot = s % WPB
        team_lo = (s // WPB) * WPB
        base = batch * N + slot * CH

        pltpu.sync_copy(imp_hbm.at[pl.ds(base, CH)], x_v)

        def keys_body(g, carry):
            for u in range(U):
                i = g * U + u
                bits = lax.bitcast_convert_type(x_v[pl.ds(i * 16, 16)],
                                                jnp.int32)
                key_v[pl.ds(i * 16, 16)] = jnp.where(
                    bits >= 0, bits, jnp.int32(_INT_MIN) - bits)
            return carry

        lax.fori_loop(0, NG, keys_body, jnp.int32(0))

        one = jnp.int32(1)
        zero16 = jnp.zeros((16,), jnp.int32)

        def publish(parity, vecs):
            for j, vec in enumerate(vecs):
                stage_v[pl.ds(j * ROW, ROW)] = vec
            off = (parity * NS + s) * (RPW * ROW)
            pltpu.sync_copy(stage_v, counts_sm.at[pl.ds(off, RPW * ROW)])
            plsc.subcore_barrier()
            toff = (parity * NS + team_lo) * (RPW * ROW)
            pltpu.sync_copy(counts_sm.at[pl.ds(toff, WPB * RPW * ROW)], team_v)

        def team_sum(j):
            def body(r, acc):
                return acc + team_v[pl.ds(r * (RPW * ROW) + j * ROW, ROW)]
            return jnp.sum(lax.fori_loop(0, WPB, body, zero16))

        def radix_body(t, basev):
            shift = jnp.int32(30) - 2 * t
            cand1 = basev + (one << shift)
            cand2 = basev + (jnp.int32(2) << shift)
            cand3 = basev + (jnp.int32(3) << shift)

            def scan(g, accs):
                a1, a2, a3 = accs
                for u in range(U):
                    kv = key_v[pl.ds((g * U + u) * 16, 16)]
                    a1 = a1 + jnp.where(kv >= cand1, one, 0)
                    a2 = a2 + jnp.where(kv >= cand2, one, 0)
                    a3 = a3 + jnp.where(kv >= cand3, one, 0)
                return a1, a2, a3

            a1, a2, a3 = lax.fori_loop(0, NG, scan, (zero16, zero16, zero16))
            publish(t % 2, [a1, a2, a3])
            t1, t2, t3 = team_sum(0), team_sum(1), team_sum(2)
            return jnp.where(
                t3 >= k, cand3,
                jnp.where(t2 >= k, cand2, jnp.where(t1 >= k, cand1, basev)))

        v = lax.fori_loop(0, 16, radix_body, jnp.int32(_INT_MIN))

        def count_scan(g, accs):
            ag, at_ = accs
            for u in range(U):
                kv = key_v[pl.ds((g * U + u) * 16, 16)]
                ag = ag + jnp.where(kv > v, one, 0)
                at_ = at_ + jnp.where(kv == v, one, 0)
            return ag, at_

        accg, acct = lax.fori_loop(0, NG, count_scan, (zero16, zero16))
        publish(0, [accg, acct])
        r_need = jnp.int32(k) - team_sum(0)
        tie_local = jnp.sum(acct)

        def prefix_body(rr, acc):
            rowsum = jnp.sum(team_v[pl.ds(rr * (RPW * ROW) + ROW, ROW)])
            return acc + jnp.where(rr < slot, rowsum, jnp.int32(0))

        tie_before = lax.fori_loop(0, WPB, prefix_body, jnp.int32(0))
        q = jnp.minimum(jnp.maximum(r_need - tie_before, jnp.int32(0)),
                        tie_local)

        fone, fzero = jnp.float32(1.0), jnp.float32(0.0)

        def write_plain(_):
            def body(g, carry):
                for u in range(U):
                    i = g * U + u
                    kv = key_v[pl.ds(i * 16, 16)]
                    out_v[pl.ds(i * 16, 16)] = jnp.where(kv > v, fone, fzero)
                return carry
            return lax.fori_loop(0, NG, body, jnp.int32(0))

        def write_all_ties(_):
            def body(g, carry):
                for u in range(U):
                    i = g * U + u
                    kv = key_v[pl.ds(i * 16, 16)]
                    out_v[pl.ds(i * 16, 16)] = jnp.where(kv >= v, fone, fzero)
                return carry
            return lax.fori_loop(0, NG, body, jnp.int32(0))

        def write_cumsum(_):
            def body(i, run):
                kv = key_v[pl.ds(i * 16, 16)]
                tie = kv == v
                csum = lax.cumsum(jnp.where(tie, one, 0))
                accept = tie & ((run + csum) <= q)
                out_v[pl.ds(i * 16, 16)] = jnp.where(
                    (kv > v) | accept, fone, fzero)
                return run + jnp.max(csum)
            return lax.fori_loop(0, NV, body, jnp.int32(0))

        _ = lax.cond(
            q == 0, write_plain,
            lambda _: lax.cond(q == tie_local, write_all_ties,
                               write_cumsum, 0),
            0)

        pltpu.sync_copy(out_v, out_hbm.at[pl.ds(base, CH)])

    return sc_mask(imp_flat)


def _sc_multiply(spikes_flat, mask, T, B, C, N):
    info = plsc.get_sparse_core_info()
    NC, NS = info.num_cores, info.num_subcores
    WPB = (NC * NS) // B
    RPW = (T * C) // WPB
    HALF = N // 2
    U = 8
    NGRP = HALF // 16 // U
    NCH = RPW * 2
    mesh = plsc.VectorSubcoreMesh(core_axis_name="c", subcore_axis_name="s")

    @functools.partial(
        pl.kernel,
        mesh=mesh,
        compiler_params=pltpu.CompilerParams(needs_layout_passes=False),
        out_type=jax.ShapeDtypeStruct((T * B * C * N,), jnp.float32),
        scratch_types=[
            pltpu.VMEM((N,), jnp.float32),
            pltpu.VMEM((HALF,), jnp.float32),
            pltpu.VMEM((HALF,), jnp.float32),
            pltpu.SemaphoreType.DMA,
            pltpu.SemaphoreType.DMA,
            pltpu.SemaphoreType.DMA,
            pltpu.SemaphoreType.DMA,
        ],
    )
    def sc_mul(spikes_hbm, mask_hbm, out_hbm, mask_v, d0, d1,
               isem0, isem1, osem0, osem1):
        c = lax.axis_index("c")
        s = lax.axis_index("s")
        batch = c * (B // NC) + s // WPB
        wslot = s % WPB

        pltpu.sync_copy(mask_hbm.at[pl.ds(batch * N, N)], mask_v)

        def off_of(ch):
            j = ch // 2
            h = ch % 2
            jj = wslot * RPW + j
            t = jj // C
            cch = jj % C
            return (((t * B) + batch) * C + cch) * N + h * HALF

        bufs = (d0, d1)
        isems = (isem0, isem1)
        osems = (osem0, osem1)

        pltpu.async_copy(
            spikes_hbm.at[pl.ds(off_of(0), HALF)], d0, isem0)

        def step(g, carry):
            for b2 in range(2):
                ch = g * 2 + b2
                me, other = bufs[b2], bufs[1 - b2]

                @pl.when(ch + 1 < NCH)
                def _start_next():
                    @pl.when(ch >= 1)
                    def _drain_other_out():
                        pltpu.make_async_copy(
                            other,
                            out_hbm.at[pl.ds(off_of(ch - 1), HALF)],
                            osems[1 - b2]).wait()
                    pltpu.async_copy(
                        spikes_hbm.at[pl.ds(off_of(ch + 1), HALF)],
                        other, isems[1 - b2])

                pltpu.make_async_copy(
                    spikes_hbm.at[pl.ds(off_of(ch), HALF)], me,
                    isems[b2]).wait()

                hoff = b2 * HALF

                def mul_body(gg, cc):
                    for u in range(U):
                        i = gg * U + u
                        me[pl.ds(i * 16, 16)] = (
                            me[pl.ds(i * 16, 16)]
                            * mask_v[pl.ds(hoff + i * 16, 16)])
                    return cc

                lax.fori_loop(0, NGRP, mul_body, jnp.int32(0))
                pltpu.async_copy(
                    me, out_hbm.at[pl.ds(off_of(ch), HALF)], osems[b2])
            return carry

        lax.fori_loop(0, NCH // 2, step, jnp.int32(0))
        pltpu.make_async_copy(
            d0, out_hbm.at[pl.ds(off_of(NCH - 2), HALF)], osem0).wait()
        pltpu.make_async_copy(
            d1, out_hbm.at[pl.ds(off_of(NCH - 1), HALF)], osem1).wait()

    return sc_mul(spikes_flat, mask)


def kernel(spikes, importance, training):
    T, B, C, H, W = spikes.shape
    N = H * W
    k = max(1, int(_TARGET_RATE * N))
    mask = _build_mask_sc(importance.reshape(B * N), B, N, k)
    out = _sc_multiply(spikes.reshape(T * B * C * N), mask, T, B, C, N)
    return out.reshape(T, B, C, H, W)


# speedup vs baseline: 1.4490x; 1.2390x over previous
import functools

import jax
import jax.numpy as jnp
from jax import lax
from jax.experimental import pallas as pl
from jax.experimental.pallas import tpu as pltpu
from jax.experimental.pallas import tpu_sc as plsc

_TARGET_RATE = 0.5
_INT_MIN = -2147483648


def _build_mask_sc(imp_flat, B, N, k):
    info = plsc.get_sparse_core_info()
    NC, NS = info.num_cores, info.num_subcores
    WPB = (NC * NS) // B
    CH = N // WPB
    NV = CH // 16
    U = 8 if NV % 8 == 0 else 1
    NG = NV // U
    ROW = 16
    RPW = 3
    mesh = plsc.VectorSubcoreMesh(core_axis_name="c", subcore_axis_name="s")

    @functools.partial(
        pl.kernel,
        mesh=mesh,
        compiler_params=pltpu.CompilerParams(needs_layout_passes=False),
        out_type=jax.ShapeDtypeStruct((B * N,), jnp.float32),
        scratch_types=[
            pltpu.VMEM((CH,), jnp.float32),
            pltpu.VMEM((CH,), jnp.int32),
            pltpu.VMEM((CH,), jnp.float32),
            pltpu.VMEM((RPW * ROW,), jnp.int32),
            pltpu.VMEM((WPB * RPW * ROW,), jnp.int32),
            pltpu.VMEM_SHARED((2 * NS * RPW * ROW,), jnp.int32),
        ],
    )
    def sc_mask(imp_hbm, out_hbm, x_v, key_v, out_v, stage_v, team_v, counts_sm):
        c = lax.axis_index("c")
        s = lax.axis_index("s")
        batch = c * (B // NC) + s // WPB
        slot = s % WPB
        team_lo = (s // WPB) * WPB
        base = batch * N + slot * CH

        pltpu.sync_copy(imp_hbm.at[pl.ds(base, CH)], x_v)

        def keys_body(g, carry):
            for u in range(U):
                i = g * U + u
                bits = lax.bitcast_convert_type(x_v[pl.ds(i * 16, 16)],
                                                jnp.int32)
                key_v[pl.ds(i * 16, 16)] = jnp.where(
                    bits >= 0, bits, jnp.int32(_INT_MIN) - bits)
            return carry

        lax.fori_loop(0, NG, keys_body, jnp.int32(0))

        one = jnp.int32(1)
        zero16 = jnp.zeros((16,), jnp.int32)

        def publish(parity, vecs):
            for j, vec in enumerate(vecs):
                stage_v[pl.ds(j * ROW, ROW)] = vec
            off = (parity * NS + s) * (RPW * ROW)
            pltpu.sync_copy(stage_v, counts_sm.at[pl.ds(off, RPW * ROW)])
            plsc.subcore_barrier()
            toff = (parity * NS + team_lo) * (RPW * ROW)
            pltpu.sync_copy(counts_sm.at[pl.ds(toff, WPB * RPW * ROW)], team_v)

        def team_sum(j):
            def body(r, acc):
                return acc + team_v[pl.ds(r * (RPW * ROW) + j * ROW, ROW)]
            return jnp.sum(lax.fori_loop(0, WPB, body, zero16))

        def radix_body(t, basev):
            shift = jnp.int32(30) - 2 * t
            cand1 = basev + (one << shift)
            cand2 = basev + (jnp.int32(2) << shift)
            cand3 = basev + (jnp.int32(3) << shift)

            def scan(g, accs):
                a1, a2, a3 = accs
                for u in range(U):
                    kv = key_v[pl.ds((g * U + u) * 16, 16)]
                    a1 = a1 + jnp.where(kv >= cand1, one, 0)
                    a2 = a2 + jnp.where(kv >= cand2, one, 0)
                    a3 = a3 + jnp.where(kv >= cand3, one, 0)
                return a1, a2, a3

            a1, a2, a3 = lax.fori_loop(0, NG, scan, (zero16, zero16, zero16))
            publish(t % 2, [a1, a2, a3])
            t1, t2, t3 = team_sum(0), team_sum(1), team_sum(2)
            return jnp.where(
                t3 >= k, cand3,
                jnp.where(t2 >= k, cand2, jnp.where(t1 >= k, cand1, basev)))

        v = lax.fori_loop(0, 16, radix_body, jnp.int32(_INT_MIN))

        def count_scan(g, accs):
            ag, at_ = accs
            for u in range(U):
                kv = key_v[pl.ds((g * U + u) * 16, 16)]
                ag = ag + jnp.where(kv > v, one, 0)
                at_ = at_ + jnp.where(kv == v, one, 0)
            return ag, at_

        accg, acct = lax.fori_loop(0, NG, count_scan, (zero16, zero16))
        publish(0, [accg, acct])
        r_need = jnp.int32(k) - team_sum(0)
        tie_local = jnp.sum(acct)

        def prefix_body(rr, acc):
            rowsum = jnp.sum(team_v[pl.ds(rr * (RPW * ROW) + ROW, ROW)])
            return acc + jnp.where(rr < slot, rowsum, jnp.int32(0))

        tie_before = lax.fori_loop(0, WPB, prefix_body, jnp.int32(0))
        q = jnp.minimum(jnp.maximum(r_need - tie_before, jnp.int32(0)),
                        tie_local)

        fone, fzero = jnp.float32(1.0), jnp.float32(0.0)

        def write_plain(_):
            def body(g, carry):
                for u in range(U):
                    i = g * U + u
                    kv = key_v[pl.ds(i * 16, 16)]
                    out_v[pl.ds(i * 16, 16)] = jnp.where(kv > v, fone, fzero)
                return carry
            return lax.fori_loop(0, NG, body, jnp.int32(0))

        def write_all_ties(_):
            def body(g, carry):
                for u in range(U):
                    i = g * U + u
                    kv = key_v[pl.ds(i * 16, 16)]
                    out_v[pl.ds(i * 16, 16)] = jnp.where(kv >= v, fone, fzero)
                return carry
            return lax.fori_loop(0, NG, body, jnp.int32(0))

        def write_cumsum(_):
            def body(i, run):
                kv = key_v[pl.ds(i * 16, 16)]
                tie = kv == v
                csum = lax.cumsum(jnp.where(tie, one, 0))
                accept = tie & ((run + csum) <= q)
                out_v[pl.ds(i * 16, 16)] = jnp.where(
                    (kv > v) | accept, fone, fzero)
                return run + jnp.max(csum)
            return lax.fori_loop(0, NV, body, jnp.int32(0))

        _ = lax.cond(
            q == 0, write_plain,
            lambda _: lax.cond(q == tie_local, write_all_ties,
                               write_cumsum, 0),
            0)

        pltpu.sync_copy(out_v, out_hbm.at[pl.ds(base, CH)])

    return sc_mask(imp_flat)


def _mul_kernel(s_ref, m_ref, o_ref):
    o_ref[...] = s_ref[...] * m_ref[...]


def kernel(spikes, importance, training):
    T, B, C, H, W = spikes.shape
    N = H * W
    k = max(1, int(_TARGET_RATE * N))

    mask = _build_mask_sc(importance.reshape(B * N), B, N, k)
    mask = mask.reshape(B, 1, N)

    s = spikes.reshape(T * B, C, N)
    cb = next(c for c in range(min(48, C), 0, -1) if C % c == 0)
    out = pl.pallas_call(
        _mul_kernel,
        grid=(T * B, C // cb),
        in_specs=[
            pl.BlockSpec((1, cb, N), lambda i, j: (i, j, 0)),
            pl.BlockSpec((1, 1, N), lambda i, j: (i % B, 0, 0)),
        ],
        out_specs=pl.BlockSpec((1, cb, N), lambda i, j: (i, j, 0)),
        out_shape=jax.ShapeDtypeStruct((T * B, C, N), jnp.float32),
        compiler_params=pltpu.CompilerParams(
            dimension_semantics=("parallel", "parallel")),
    )(s, mask)
    return out.reshape(T, B, C, H, W)


# speedup vs baseline: 2.1076x; 1.4545x over previous
import functools

import jax
import jax.numpy as jnp
from jax import lax
from jax.experimental import pallas as pl
from jax.experimental.pallas import tpu as pltpu
from jax.experimental.pallas import tpu_sc as plsc

_TARGET_RATE = 0.5
_INT_MIN = -2147483648


def _build_mask_sc(imp_flat, B, N, k):
    info = plsc.get_sparse_core_info()
    NC, NS = info.num_cores, info.num_subcores
    WPB = (NC * NS) // B
    CH = N // WPB
    NV = CH // 16
    U = 8 if NV % 8 == 0 else 1
    NG = NV // U
    ROW = 16
    RPW = 3
    mesh = plsc.VectorSubcoreMesh(core_axis_name="c", subcore_axis_name="s")

    @functools.partial(
        pl.kernel,
        mesh=mesh,
        compiler_params=pltpu.CompilerParams(needs_layout_passes=False),
        out_type=jax.ShapeDtypeStruct((B * N,), jnp.float32),
        scratch_types=[
            pltpu.VMEM((CH,), jnp.float32),
            pltpu.VMEM((CH,), jnp.int32),
            pltpu.VMEM((CH,), jnp.float32),
            pltpu.VMEM((RPW * ROW,), jnp.int32),
            pltpu.VMEM((WPB * RPW * ROW,), jnp.int32),
            pltpu.VMEM_SHARED((2 * NS * RPW * ROW,), jnp.int32),
        ],
    )
    def sc_mask(imp_hbm, out_hbm, x_v, key_v, out_v, stage_v, team_v, counts_sm):
        c = lax.axis_index("c")
        s = lax.axis_index("s")
        batch = c * (B // NC) + s // WPB
        slot = s % WPB
        team_lo = (s // WPB) * WPB
        base = batch * N + slot * CH

        pltpu.sync_copy(imp_hbm.at[pl.ds(base, CH)], x_v)

        def keys_body(g, carry):
            for u in range(U):
                i = g * U + u
                bits = lax.bitcast_convert_type(x_v[pl.ds(i * 16, 16)],
                                                jnp.int32)
                key_v[pl.ds(i * 16, 16)] = jnp.where(
                    bits >= 0, bits, jnp.int32(_INT_MIN) - bits)
            return carry

        lax.fori_loop(0, NG, keys_body, jnp.int32(0))

        one = jnp.int32(1)
        zero16 = jnp.zeros((16,), jnp.int32)

        def publish(parity, vecs):
            for j, vec in enumerate(vecs):
                stage_v[pl.ds(j * ROW, ROW)] = vec
            off = (parity * NS + s) * (RPW * ROW)
            pltpu.sync_copy(stage_v, counts_sm.at[pl.ds(off, RPW * ROW)])
            plsc.subcore_barrier()
            toff = (parity * NS + team_lo) * (RPW * ROW)
            pltpu.sync_copy(counts_sm.at[pl.ds(toff, WPB * RPW * ROW)], team_v)

        def team_sum(j):
            def body(r, acc):
                return acc + team_v[pl.ds(r * (RPW * ROW) + j * ROW, ROW)]
            return jnp.sum(lax.fori_loop(0, WPB, body, zero16))

        def radix_body(t, basev):
            shift = jnp.int32(30) - 2 * t
            cand1 = basev + (one << shift)
            cand2 = basev + (jnp.int32(2) << shift)
            cand3 = basev + (jnp.int32(3) << shift)

            def scan(g, accs):
                a1, a2, a3 = accs
                for u in range(U):
                    kv = key_v[pl.ds((g * U + u) * 16, 16)]
                    a1 = a1 + jnp.where(kv >= cand1, one, 0)
                    a2 = a2 + jnp.where(kv >= cand2, one, 0)
                    a3 = a3 + jnp.where(kv >= cand3, one, 0)
                return a1, a2, a3

            a1, a2, a3 = lax.fori_loop(0, NG, scan, (zero16, zero16, zero16))
            publish(t % 2, [a1, a2, a3])
            t1, t2, t3 = team_sum(0), team_sum(1), team_sum(2)
            return jnp.where(
                t3 >= k, cand3,
                jnp.where(t2 >= k, cand2, jnp.where(t1 >= k, cand1, basev)))

        v = lax.fori_loop(0, 16, radix_body, jnp.int32(_INT_MIN))

        def count_scan(g, accs):
            ag, at_ = accs
            for u in range(U):
                kv = key_v[pl.ds((g * U + u) * 16, 16)]
                ag = ag + jnp.where(kv > v, one, 0)
                at_ = at_ + jnp.where(kv == v, one, 0)
            return ag, at_

        accg, acct = lax.fori_loop(0, NG, count_scan, (zero16, zero16))
        publish(0, [accg, acct])
        r_need = jnp.int32(k) - team_sum(0)
        tie_local = jnp.sum(acct)

        def prefix_body(rr, acc):
            rowsum = jnp.sum(team_v[pl.ds(rr * (RPW * ROW) + ROW, ROW)])
            return acc + jnp.where(rr < slot, rowsum, jnp.int32(0))

        tie_before = lax.fori_loop(0, WPB, prefix_body, jnp.int32(0))
        q = jnp.minimum(jnp.maximum(r_need - tie_before, jnp.int32(0)),
                        tie_local)

        fone, fzero = jnp.float32(1.0), jnp.float32(0.0)

        def write_plain(_):
            def body(g, carry):
                for u in range(U):
                    i = g * U + u
                    kv = key_v[pl.ds(i * 16, 16)]
                    out_v[pl.ds(i * 16, 16)] = jnp.where(kv > v, fone, fzero)
                return carry
            return lax.fori_loop(0, NG, body, jnp.int32(0))

        def write_all_ties(_):
            def body(g, carry):
                for u in range(U):
                    i = g * U + u
                    kv = key_v[pl.ds(i * 16, 16)]
                    out_v[pl.ds(i * 16, 16)] = jnp.where(kv >= v, fone, fzero)
                return carry
            return lax.fori_loop(0, NG, body, jnp.int32(0))

        def write_cumsum(_):
            def body(i, run):
                kv = key_v[pl.ds(i * 16, 16)]
                tie = kv == v
                csum = lax.cumsum(jnp.where(tie, one, 0))
                accept = tie & ((run + csum) <= q)
                out_v[pl.ds(i * 16, 16)] = jnp.where(
                    (kv > v) | accept, fone, fzero)
                return run + jnp.max(csum)
            return lax.fori_loop(0, NV, body, jnp.int32(0))

        _ = lax.cond(
            q == 0, write_plain,
            lambda _: lax.cond(q == tie_local, write_all_ties,
                               write_cumsum, 0),
            0)

        pltpu.sync_copy(out_v, out_hbm.at[pl.ds(base, CH)])

    return sc_mask(imp_flat)


def _sc_multiply(spikes3, mask3, T, B, C, H, W):
    info = plsc.get_sparse_core_info()
    NC, NS = info.num_cores, info.num_subcores
    WPB = (NC * NS) // B
    RPW = (T * C) // WPB
    HH = H // 2
    WCH = W // 16
    NCH = RPW * 2
    mesh = plsc.VectorSubcoreMesh(core_axis_name="c", subcore_axis_name="s")

    @functools.partial(
        pl.kernel,
        mesh=mesh,
        compiler_params=pltpu.CompilerParams(needs_layout_passes=False),
        out_type=jax.ShapeDtypeStruct((T * B * C, H, W), jnp.float32),
        scratch_types=[
            pltpu.VMEM((H, W), jnp.float32),
            pltpu.VMEM((HH, W), jnp.float32),
            pltpu.VMEM((HH, W), jnp.float32),
            pltpu.SemaphoreType.DMA,
            pltpu.SemaphoreType.DMA,
            pltpu.SemaphoreType.DMA,
            pltpu.SemaphoreType.DMA,
        ],
    )
    def sc_mul(spikes_hbm, mask_hbm, out_hbm, mask_v, d0, d1,
               isem0, isem1, osem0, osem1):
        c = lax.axis_index("c")
        s = lax.axis_index("s")
        batch = c * (B // NC) + s // WPB
        wslot = s % WPB

        pltpu.sync_copy(mask_hbm.at[batch], mask_v)

        def slice_of(ch):
            j = ch // 2
            h2 = ch % 2
            jj = wslot * RPW + j
            t = jj // C
            cch = jj % C
            row = ((t * B) + batch) * C + cch
            return row, h2 * HH

        bufs = (d0, d1)
        isems = (isem0, isem1)
        osems = (osem0, osem1)

        r0, h0 = slice_of(0)
        pltpu.async_copy(
            spikes_hbm.at[r0, pl.ds(h0, HH), :], d0, isem0)

        def step(g, carry):
            for b2 in range(2):
                ch = g * 2 + b2
                me, other = bufs[b2], bufs[1 - b2]

                @pl.when(ch + 1 < NCH)
                def _start_next():
                    rn, hn = slice_of(ch + 1)

                    @pl.when(ch >= 1)
                    def _drain_other_out():
                        rp, hp = slice_of(ch - 1)
                        pltpu.make_async_copy(
                            other,
                            out_hbm.at[rp, pl.ds(hp, HH), :],
                            osems[1 - b2]).wait()
                    pltpu.async_copy(
                        spikes_hbm.at[rn, pl.ds(hn, HH), :],
                        other, isems[1 - b2])

                rc, hc = slice_of(ch)
                pltpu.make_async_copy(
                    spikes_hbm.at[rc, pl.ds(hc, HH), :], me,
                    isems[b2]).wait()

                moff = b2 * HH

                def mul_body(hh, cc):
                    for u in range(WCH):
                        me[hh, pl.ds(u * 16, 16)] = (
                            me[hh, pl.ds(u * 16, 16)]
                            * mask_v[moff + hh, pl.ds(u * 16, 16)])
                    return cc

                lax.fori_loop(0, HH, mul_body, jnp.int32(0))
                pltpu.async_copy(
                    me, out_hbm.at[rc, pl.ds(hc, HH), :], osems[b2])
            return carry

        lax.fori_loop(0, NCH // 2, step, jnp.int32(0))
        rz, hz = slice_of(NCH - 2)
        pltpu.make_async_copy(
            d0, out_hbm.at[rz, pl.ds(hz, HH), :], osem0).wait()
        ry, hy = slice_of(NCH - 1)
        pltpu.make_async_copy(
            d1, out_hbm.at[ry, pl.ds(hy, HH), :], osem1).wait()

    return sc_mul(spikes3, mask3)


def kernel(spikes, importance, training):
    T, B, C, H, W = spikes.shape
    N = H * W
    k = max(1, int(_TARGET_RATE * N))
    mask = _build_mask_sc(importance.reshape(B * N), B, N, k)
    out = _sc_multiply(spikes.reshape(T * B * C, H, W),
                       mask.reshape(B, H, W), T, B, C, H, W)
    return out.reshape(T, B, C, H, W)
